# Initial kernel scaffold; baseline (speedup 1.0000x reference)
import jax
import jax.numpy as jnp
from jax.experimental import pallas as pl


def kernel(node_features, edge_index, edge_type, W, W_root, bias):
    raise NotImplementedError("write your pallas kernel here")


# speedup vs baseline: 3.5991x; 3.5991x over previous
import functools

import jax
import jax.numpy as jnp
from jax import lax
from jax.experimental import pallas as pl
from jax.experimental.pallas import tpu as pltpu
from jax.experimental.pallas import tpu_sc as plsc

_NC = 2
_NS = 16
_NW = _NC * _NS
_LANES = 16


def _make_mesh():
  return plsc.VectorSubcoreMesh(core_axis_name="c", subcore_axis_name="s",
                                num_cores=_NC, num_subcores=_NS)


def _make_binning_kernel(E, R, chunk, nchunk, cap):
  ew = E // _NW
  assert ew * _NW == E and chunk * nchunk == ew
  nseg = _NW * nchunk

  @functools.partial(
      pl.kernel,
      mesh=_make_mesh(),
      compiler_params=pltpu.CompilerParams(needs_layout_passes=False),
      out_type=[
          jax.ShapeDtypeStruct((nseg * R * cap,), jnp.int32),
          jax.ShapeDtypeStruct((nseg * R * cap,), jnp.int32),
          jax.ShapeDtypeStruct((nseg * _LANES,), jnp.int32),
      ],
      scratch_types=[
          pltpu.VMEM((chunk,), jnp.int32),
          pltpu.VMEM((chunk,), jnp.int32),
          pltpu.VMEM((chunk,), jnp.int32),
      ] + [pltpu.VMEM((chunk,), jnp.int32) for _ in range(2 * R)] + [
          pltpu.VMEM((_LANES,), jnp.int32),
      ],
  )
  def binning(src_hbm, dst_hbm, et_hbm, bsrc_hbm, bdst_hbm, cnt_hbm,
              s_src, s_dst, s_et, *rest):
    b_src = rest[:R]
    b_dst = rest[R:2 * R]
    c_buf = rest[2 * R]
    wid = lax.axis_index("s") * _NC + lax.axis_index("c")
    iota = lax.broadcasted_iota(jnp.int32, (_LANES,), 0)

    def chunk_body(c, carry):
      seg = wid * nchunk + c
      base = wid * ew + c * chunk
      pltpu.sync_copy(src_hbm.at[pl.ds(base, chunk)], s_src)
      pltpu.sync_copy(dst_hbm.at[pl.ds(base, chunk)], s_dst)
      pltpu.sync_copy(et_hbm.at[pl.ds(base, chunk)], s_et)

      def vec_body(v, cnt_vec):
        t16 = s_et[pl.ds(v * _LANES, _LANES)]
        sv = s_src[pl.ds(v * _LANES, _LANES)]
        dv = s_dst[pl.ds(v * _LANES, _LANES)]
        for r in range(R):
          m = t16 == r
          mi = m.astype(jnp.int32)
          incl = plsc.cumsum(mi)
          off_r = jnp.sum(jnp.where(iota == r, cnt_vec, 0))
          pos = incl + (off_r - 1)
          plsc.store_scatter(b_src[r], [pos], sv, mask=m)
          plsc.store_scatter(b_dst[r], [pos], dv, mask=m)
          tot = jnp.sum(mi)
          cnt_vec = cnt_vec + jnp.where(iota == r, tot, 0)
        return cnt_vec

      cnt_vec = lax.fori_loop(0, chunk // _LANES, vec_body,
                              jnp.zeros((_LANES,), jnp.int32))
      for r in range(R):
        pltpu.sync_copy(b_src[r], bsrc_hbm.at[pl.ds((seg * R + r) * cap, chunk)])
        pltpu.sync_copy(b_dst[r], bdst_hbm.at[pl.ds((seg * R + r) * cap, chunk)])
      c_buf[...] = cnt_vec
      pltpu.sync_copy(c_buf, cnt_hbm.at[pl.ds(seg * _LANES, _LANES)])
      return carry

    lax.fori_loop(0, nchunk, chunk_body, 0)

  return binning


def _make_agg_kernel(N, D, R, nseg, cap, dpt, sub):
  ndb = D // dpt
  rounds = (R * ndb) // _NW
  assert rounds * _NW == R * ndb

  @functools.partial(
      pl.kernel,
      mesh=_make_mesh(),
      compiler_params=pltpu.CompilerParams(needs_layout_passes=False),
      out_type=[
          jax.ShapeDtypeStruct((R, D, N), jnp.float32),
          jax.ShapeDtypeStruct((R, N), jnp.float32),
      ],
      scratch_types=[pltpu.VMEM((N,), jnp.float32) for _ in range(2 * dpt)] + [
          pltpu.VMEM((N,), jnp.float32),
          pltpu.VMEM((nseg * _LANES,), jnp.int32),
          pltpu.VMEM((sub,), jnp.int32),
          pltpu.VMEM((sub,), jnp.int32),
      ],
  )
  def agg(xt_hbm, bsrc_hbm, bdst_hbm, cnt_hbm, acct_hbm, cntf_hbm, *rest):
    xblk = rest[:dpt]
    acc = rest[dpt:2 * dpt]
    cntacc, cnts_all, e_src, e_dst = rest[2 * dpt:]
    wid = lax.axis_index("s") * _NC + lax.axis_index("c")
    iota = lax.broadcasted_iota(jnp.int32, (_LANES,), 0)
    zeros16 = jnp.zeros((_LANES,), jnp.float32)
    ones16 = jnp.ones((_LANES,), jnp.float32)
    pltpu.sync_copy(cnt_hbm, cnts_all)

    def round_body(ro, carry):
      t = ro * _NW + wid
      dblock = t // R
      rr = (t % R + dblock) % R
      for f in range(dpt):
        pltpu.sync_copy(xt_hbm.at[dblock * dpt + f], xblk[f])

      def z_body(i, c2):
        for f in range(dpt):
          acc[f][pl.ds(i * _LANES, _LANES)] = zeros16
        cntacc[pl.ds(i * _LANES, _LANES)] = zeros16
        return c2

      lax.fori_loop(0, N // _LANES, z_body, 0)

      def seg_body(sg, c2):
        row = cnts_all[pl.ds(sg * _LANES, _LANES)]
        cnt = jnp.sum(jnp.where(iota == rr, row, 0))
        nsubchunks = (cnt + (sub - 1)) // sub
        segbase = (sg * R + rr) * cap

        def sub_body(b, c3):
          pltpu.sync_copy(bsrc_hbm.at[pl.ds(segbase + b * sub, sub)], e_src)
          pltpu.sync_copy(bdst_hbm.at[pl.ds(segbase + b * sub, sub)], e_dst)
          rem = cnt - b * sub

          def v_body(v, c4):
            m = (iota + v * _LANES) < rem
            sv = e_src[pl.ds(v * _LANES, _LANES)]
            dv = e_dst[pl.ds(v * _LANES, _LANES)]
            for f in range(dpt):
              vals = plsc.load_gather(xblk[f], [sv], mask=m)
              plsc.addupdate_scatter(acc[f], [dv], vals, mask=m)
            plsc.addupdate_scatter(cntacc, [dv], ones16, mask=m)
            return c4

          lax.fori_loop(0, sub // _LANES, v_body, 0)
          return c3

        lax.fori_loop(0, nsubchunks, sub_body, 0)
        return c2

      lax.fori_loop(0, nseg, seg_body, 0)
      for f in range(dpt):
        pltpu.sync_copy(acc[f], acct_hbm.at[rr, dblock * dpt + f])

      @pl.when(dblock == 0)
      def _():
        pltpu.sync_copy(cntacc, cntf_hbm.at[rr])

      return carry

    lax.fori_loop(0, rounds, round_body, 0)

  return agg


def _make_combine_kernel(N, D, R, bn):
  assert N % bn == 0

  def body(x_ref, acc_ref, cnt_ref, w_ref, wr_ref, b_ref, out_ref):
    x = x_ref[...]
    out = jnp.dot(x, wr_ref[...], preferred_element_type=jnp.float32)
    out = out + b_ref[...]
    for r in range(R):
      a = acc_ref[r]
      c = cnt_ref[pl.ds(r, 1), :]
      inv = 1.0 / jnp.maximum(c, 1.0)
      out = out + lax.dot_general(
          a * inv, w_ref[r], (((0,), (0,)), ((), ())),
          preferred_element_type=jnp.float32)
    out_ref[...] = jnp.maximum(out, 0.0)

  return pl.pallas_call(
      body,
      grid=(N // bn,),
      in_specs=[
          pl.BlockSpec((bn, D), lambda i: (i, 0)),
          pl.BlockSpec((R, D, bn), lambda i: (0, 0, i)),
          pl.BlockSpec((R, bn), lambda i: (0, i)),
          pl.BlockSpec((R, D, D), lambda i: (0, 0, 0)),
          pl.BlockSpec((D, D), lambda i: (0, 0)),
          pl.BlockSpec((1, D), lambda i: (0, 0)),
      ],
      out_specs=pl.BlockSpec((bn, D), lambda i: (i, 0)),
      out_shape=jax.ShapeDtypeStruct((N, D), jnp.float32),
  )


def kernel(node_features, edge_index, edge_type, W, W_root, bias):
  N, D = node_features.shape
  E = edge_index.shape[1]
  R = W.shape[0]

  chunk = 2000
  nchunk = (E // _NW) // chunk
  cap = 2048
  nseg = _NW * nchunk
  dpt = 4
  sub = 512
  bn = 1024
  npad = ((N + bn - 1) // bn) * bn

  src = edge_index[0]
  dst = edge_index[1]
  xp = jnp.pad(node_features, ((0, npad - N), (0, 0)))
  xt = xp.T

  binning = _make_binning_kernel(E, R, chunk, nchunk, cap)
  bsrc, bdst, cnts = binning(src, dst, edge_type)

  agg = _make_agg_kernel(npad, D, R, nseg, cap, dpt, sub)
  acct, cntf = agg(xt, bsrc, bdst, cnts)

  combine = _make_combine_kernel(npad, D, R, bn)
  out = combine(xp, acct, cntf, W, W_root,
                bias.reshape(1, D).astype(jnp.float32))
  return out[:N]


# speedup vs baseline: 12.3201x; 3.4231x over previous
import functools

import jax
import jax.numpy as jnp
from jax import lax
from jax.experimental import pallas as pl
from jax.experimental.pallas import tpu as pltpu
from jax.experimental.pallas import tpu_sc as plsc

_NC = 2
_NS = 16
_NW = _NC * _NS
_LANES = 16


def _make_mesh():
  return plsc.VectorSubcoreMesh(core_axis_name="c", subcore_axis_name="s",
                                num_cores=_NC, num_subcores=_NS)


def _make_binning_kernel(E, R, chunk, nchunk, cap, sub, fast, dump):
  ew = E // _NW
  assert ew * _NW == E and chunk * nchunk == ew
  assert fast % sub == 0 and cap >= chunk + sub and cap % sub == 0
  nseg = _NW * nchunk
  buf = chunk + sub

  @functools.partial(
      pl.kernel,
      mesh=_make_mesh(),
      compiler_params=pltpu.CompilerParams(needs_layout_passes=False),
      out_type=[
          jax.ShapeDtypeStruct((nseg * R * cap,), jnp.int32),
          jax.ShapeDtypeStruct((nseg * R * cap,), jnp.int32),
          jax.ShapeDtypeStruct((nseg * _LANES,), jnp.int32),
      ],
      scratch_types=[
          pltpu.VMEM((ew + _LANES,), jnp.int32),
          pltpu.VMEM((ew + _LANES,), jnp.int32),
          pltpu.VMEM((ew + _LANES,), jnp.int32),
      ] + [pltpu.VMEM((buf,), jnp.int32) for _ in range(2 * R)] + [
          pltpu.VMEM((_LANES,), jnp.int32),
      ],
  )
  def binning(src_hbm, dst_hbm, et_hbm, bsrc_hbm, bdst_hbm, cnt_hbm,
              s_src, s_dst, s_et, *rest):
    b_src = rest[:R]
    b_dst = rest[R:2 * R]
    c_buf = rest[2 * R]
    wid = lax.axis_index("s") * _NC + lax.axis_index("c")
    iota = lax.broadcasted_iota(jnp.int32, (_LANES,), 0)
    zero16 = jnp.zeros((_LANES,), jnp.int32)
    dump16 = jnp.full((_LANES,), dump, jnp.int32)
    base = wid * ew
    pltpu.sync_copy(src_hbm.at[pl.ds(base, ew)], s_src.at[pl.ds(0, ew)])
    pltpu.sync_copy(dst_hbm.at[pl.ds(base, ew)], s_dst.at[pl.ds(0, ew)])
    pltpu.sync_copy(et_hbm.at[pl.ds(base, ew)], s_et.at[pl.ds(0, ew)])
    nvec = (chunk + _LANES - 1) // _LANES

    def chunk_body(c, carry):
      seg = wid * nchunk + c
      cb = c * chunk

      def vec_body(v, offs):
        t16 = s_et[pl.ds(cb + v * _LANES, _LANES)]
        sv = s_src[pl.ds(cb + v * _LANES, _LANES)]
        dv = s_dst[pl.ds(cb + v * _LANES, _LANES)]
        valid = (v * _LANES + iota) < chunk
        new_offs = []
        for r in range(R):
          m = jnp.logical_and(t16 == r, valid)
          mi = m.astype(jnp.int32)
          incl = plsc.cumsum(mi)
          pos = incl + (offs[r] - 1)
          plsc.store_scatter(b_src[r], [pos], sv, mask=m)
          plsc.store_scatter(b_dst[r], [pos], dv, mask=m)
          new_offs.append(offs[r] + jnp.sum(mi))
        return tuple(new_offs)

      offs = lax.fori_loop(0, nvec, vec_body,
                           tuple(jnp.int32(0) for _ in range(R)))

      cnts_row = zero16
      for r in range(R):
        cnt_s = offs[r]
        for k in range(sub // _LANES):
          b_src[r][pl.ds(cnt_s + k * _LANES, _LANES)] = zero16
          b_dst[r][pl.ds(cnt_s + k * _LANES, _LANES)] = dump16
        segbase = (seg * R + r) * cap
        pltpu.sync_copy(b_src[r].at[pl.ds(0, fast)],
                        bsrc_hbm.at[pl.ds(segbase, fast)])
        pltpu.sync_copy(b_dst[r].at[pl.ds(0, fast)],
                        bdst_hbm.at[pl.ds(segbase, fast)])
        nsub = (cnt_s + (sub - 1)) // sub

        def tail_body(b, c2):
          pltpu.sync_copy(b_src[r].at[pl.ds(b * sub, sub)],
                          bsrc_hbm.at[pl.ds(segbase + b * sub, sub)])
          pltpu.sync_copy(b_dst[r].at[pl.ds(b * sub, sub)],
                          bdst_hbm.at[pl.ds(segbase + b * sub, sub)])
          return c2

        lax.fori_loop(fast // sub, nsub, tail_body, 0)
        cnts_row = cnts_row + jnp.where(iota == r, offs[r], 0)

      c_buf[...] = cnts_row
      pltpu.sync_copy(c_buf, cnt_hbm.at[pl.ds(seg * _LANES, _LANES)])
      return carry

    lax.fori_loop(0, nchunk, chunk_body, 0)

  return binning


def _make_agg_kernel(N, D, R, nseg, cap, dpt, sub, fast):
  ndb = D // dpt
  rounds = (R * ndb) // _NW
  assert rounds * _NW == R * ndb and fast % sub == 0

  @functools.partial(
      pl.kernel,
      mesh=_make_mesh(),
      compiler_params=pltpu.CompilerParams(needs_layout_passes=False),
      out_type=[
          jax.ShapeDtypeStruct((R, D, N), jnp.float32),
          jax.ShapeDtypeStruct((R, N), jnp.float32),
      ],
      scratch_types=[pltpu.VMEM((N,), jnp.float32) for _ in range(2 * dpt)] + [
          pltpu.VMEM((N,), jnp.float32),
          pltpu.VMEM((nseg * _LANES,), jnp.int32),
      ] + [pltpu.VMEM((fast,), jnp.int32) for _ in range(4)] + [
          pltpu.VMEM((sub,), jnp.int32),
          pltpu.VMEM((sub,), jnp.int32),
          pltpu.SemaphoreType.DMA,
          pltpu.SemaphoreType.DMA,
      ],
  )
  def agg(xt_hbm, bsrc_hbm, bdst_hbm, cnt_hbm, acct_hbm, cntf_hbm, *rest):
    xblk = rest[:dpt]
    acc = rest[dpt:2 * dpt]
    (cntacc, cnts_all, es0, es1, ed0, ed1, t_src, t_dst,
     sem0, sem1) = rest[2 * dpt:]
    e_src = (es0, es1)
    e_dst = (ed0, ed1)
    sems = (sem0, sem1)
    wid = lax.axis_index("s") * _NC + lax.axis_index("c")
    iota = lax.broadcasted_iota(jnp.int32, (_LANES,), 0)
    zeros16 = jnp.zeros((_LANES,), jnp.float32)
    ones16 = jnp.ones((_LANES,), jnp.float32)
    pltpu.sync_copy(cnt_hbm, cnts_all)

    def round_body(ro, carry):
      t = ro * _NW + wid
      dblock = t // R
      rr = (t % R + dblock) % R
      for f in range(dpt):
        pltpu.sync_copy(xt_hbm.at[dblock * dpt + f], xblk[f])

      def z_body(i, c2):
        for f in range(dpt):
          acc[f][pl.ds(i * _LANES, _LANES)] = zeros16
        cntacc[pl.ds(i * _LANES, _LANES)] = zeros16
        return c2

      lax.fori_loop(0, N // _LANES, z_body, 0)

      def start(sg):
        slot = sg % 2
        segbase = (sg * R + rr) * cap
        c1 = pltpu.async_copy(bsrc_hbm.at[pl.ds(segbase, fast)],
                              e_src[slot], sems[slot])
        c2 = pltpu.async_copy(bdst_hbm.at[pl.ds(segbase, fast)],
                              e_dst[slot], sems[slot])
        return c1, c2

      def vreg_batch(sv_ref, dv_ref, v, base):
        sv = sv_ref[pl.ds(base + v * _LANES, _LANES)]
        dv = dv_ref[pl.ds(base + v * _LANES, _LANES)]
        for f in range(dpt):
          vals = plsc.load_gather(xblk[f], [sv])
          plsc.addupdate_scatter(acc[f], [dv], vals)
        plsc.addupdate_scatter(cntacc, [dv], ones16)

      cps = start(0)
      for sg in range(nseg):
        slot = sg % 2
        if sg + 1 < nseg:
          nxt = start(sg + 1)
        cps[0].wait()
        cps[1].wait()
        if sg + 1 < nseg:
          cps = nxt
        row = cnts_all[pl.ds(sg * _LANES, _LANES)]
        cnt = jnp.sum(jnp.where(iota == rr, row, 0))
        nv = (jnp.minimum(cnt, fast) + (_LANES - 1)) // _LANES

        def v_body(v, c4):
          vreg_batch(e_src[slot], e_dst[slot], v, 0)
          return c4

        lax.fori_loop(0, nv, v_body, 0)

        nsubchunks = (cnt + (sub - 1)) // sub
        segbase = (sg * R + rr) * cap

        def tail_body(b, c3):
          pltpu.sync_copy(bsrc_hbm.at[pl.ds(segbase + b * sub, sub)], t_src)
          pltpu.sync_copy(bdst_hbm.at[pl.ds(segbase + b * sub, sub)], t_dst)

          def tv_body(v, c4):
            vreg_batch(t_src, t_dst, v, 0)
            return c4

          lax.fori_loop(0, sub // _LANES, tv_body, 0)
          return c3

        lax.fori_loop(fast // sub, nsubchunks, tail_body, 0)

      for f in range(dpt):
        pltpu.sync_copy(acc[f], acct_hbm.at[rr, dblock * dpt + f])

      @pl.when(dblock == 0)
      def _():
        pltpu.sync_copy(cntacc, cntf_hbm.at[rr])

      return carry

    lax.fori_loop(0, rounds, round_body, 0)

  return agg


def _make_combine_kernel(N, D, R, bn):
  assert N % bn == 0

  def body(x_ref, acc_ref, cnt_ref, w_ref, wr_ref, b_ref, out_ref):
    x = x_ref[...]
    out = jnp.dot(x, wr_ref[...], preferred_element_type=jnp.float32)
    out = out + b_ref[...]
    for r in range(R):
      a = acc_ref[r]
      c = cnt_ref[pl.ds(r, 1), :]
      inv = 1.0 / jnp.maximum(c, 1.0)
      out = out + lax.dot_general(
          a * inv, w_ref[r], (((0,), (0,)), ((), ())),
          preferred_element_type=jnp.float32)
    out_ref[...] = jnp.maximum(out, 0.0)

  return pl.pallas_call(
      body,
      grid=(N // bn,),
      in_specs=[
          pl.BlockSpec((bn, D), lambda i: (i, 0)),
          pl.BlockSpec((R, D, bn), lambda i: (0, 0, i)),
          pl.BlockSpec((R, bn), lambda i: (0, i)),
          pl.BlockSpec((R, D, D), lambda i: (0, 0, 0)),
          pl.BlockSpec((D, D), lambda i: (0, 0)),
          pl.BlockSpec((1, D), lambda i: (0, 0)),
      ],
      out_specs=pl.BlockSpec((bn, D), lambda i: (i, 0)),
      out_shape=jax.ShapeDtypeStruct((N, D), jnp.float32),
  )


def kernel(node_features, edge_index, edge_type, W, W_root, bias):
  N, D = node_features.shape
  E = edge_index.shape[1]
  R = W.shape[0]

  chunk = 5000
  nchunk = (E // _NW) // chunk
  cap = 5632
  nseg = _NW * nchunk
  dpt = 4
  sub = 512
  fast = 1024
  bn = 1024
  npad = ((N + bn - 1) // bn) * bn

  src = edge_index[0]
  dst = edge_index[1]
  xp = jnp.pad(node_features, ((0, npad - N), (0, 0)))
  xt = xp.T

  binning = _make_binning_kernel(E, R, chunk, nchunk, cap, sub, fast, N)
  bsrc, bdst, cnts = binning(src, dst, edge_type)

  agg = _make_agg_kernel(npad, D, R, nseg, cap, dpt, sub, fast)
  acct, cntf = agg(xt, bsrc, bdst, cnts)

  combine = _make_combine_kernel(npad, D, R, bn)
  out = combine(xp, acct, cntf, W, W_root,
                bias.reshape(1, D).astype(jnp.float32))
  return out[:N]


# speedup vs baseline: 19.2733x; 1.5644x over previous
import functools

import jax
import jax.numpy as jnp
from jax import lax
from jax.experimental import pallas as pl
from jax.experimental.pallas import tpu as pltpu
from jax.experimental.pallas import tpu_sc as plsc

_NC = 2
_NS = 16
_NW = _NC * _NS
_LANES = 16


def _make_mesh():
  return plsc.VectorSubcoreMesh(core_axis_name="c", subcore_axis_name="s",
                                num_cores=_NC, num_subcores=_NS)


def _make_binning_kernel(E, R, chunk, nchunk, cap, sub, fast, dump):
  ew = E // _NW
  assert ew * _NW == E and chunk * nchunk == ew
  assert fast % sub == 0 and cap >= chunk + sub and cap % sub == 0
  nseg = _NW * nchunk
  buf = chunk + sub

  @functools.partial(
      pl.kernel,
      mesh=_make_mesh(),
      compiler_params=pltpu.CompilerParams(needs_layout_passes=False),
      out_type=[
          jax.ShapeDtypeStruct(((nseg + 2) * R * cap,), jnp.int32),
          jax.ShapeDtypeStruct(((nseg + 2) * R * cap,), jnp.int32),
          jax.ShapeDtypeStruct((nseg * _LANES,), jnp.int32),
      ],
      scratch_types=[
          pltpu.VMEM((ew + _LANES,), jnp.int32),
          pltpu.VMEM((ew + _LANES,), jnp.int32),
          pltpu.VMEM((ew + _LANES,), jnp.int32),
      ] + [pltpu.VMEM((buf,), jnp.int32) for _ in range(2 * R)] + [
          pltpu.VMEM((_LANES,), jnp.int32),
      ],
  )
  def binning(src_hbm, dst_hbm, et_hbm, bsrc_hbm, bdst_hbm, cnt_hbm,
              s_src, s_dst, s_et, *rest):
    b_src = rest[:R]
    b_dst = rest[R:2 * R]
    c_buf = rest[2 * R]
    wid = lax.axis_index("s") * _NC + lax.axis_index("c")
    iota = lax.broadcasted_iota(jnp.int32, (_LANES,), 0)
    zero16 = jnp.zeros((_LANES,), jnp.int32)
    dump16 = jnp.full((_LANES,), dump, jnp.int32)
    base = wid * ew
    pltpu.sync_copy(src_hbm.at[pl.ds(base, ew)], s_src.at[pl.ds(0, ew)])
    pltpu.sync_copy(dst_hbm.at[pl.ds(base, ew)], s_dst.at[pl.ds(0, ew)])
    pltpu.sync_copy(et_hbm.at[pl.ds(base, ew)], s_et.at[pl.ds(0, ew)])
    nvec = (chunk + _LANES - 1) // _LANES

    def chunk_body(c, carry):
      seg = wid * nchunk + c
      cb = c * chunk

      def vec_body(v, offs):
        t16 = s_et[pl.ds(cb + v * _LANES, _LANES)]
        sv = s_src[pl.ds(cb + v * _LANES, _LANES)]
        dv = s_dst[pl.ds(cb + v * _LANES, _LANES)]
        valid = (v * _LANES + iota) < chunk
        new_offs = []
        for r in range(R):
          m = jnp.logical_and(t16 == r, valid)
          mi = m.astype(jnp.int32)
          incl = plsc.cumsum(mi)
          pos = incl + (offs[r] - 1)
          plsc.store_scatter(b_src[r], [pos], sv, mask=m)
          plsc.store_scatter(b_dst[r], [pos], dv, mask=m)
          new_offs.append(offs[r] + jnp.sum(mi))
        return tuple(new_offs)

      offs = lax.fori_loop(0, nvec, vec_body,
                           tuple(jnp.int32(0) for _ in range(R)))

      cnts_row = zero16
      for r in range(R):
        cnt_s = offs[r]
        for k in range(sub // _LANES):
          b_src[r][pl.ds(cnt_s + k * _LANES, _LANES)] = zero16
          b_dst[r][pl.ds(cnt_s + k * _LANES, _LANES)] = dump16
        segbase = (seg * R + r) * cap
        pltpu.sync_copy(b_src[r].at[pl.ds(0, fast)],
                        bsrc_hbm.at[pl.ds(segbase, fast)])
        pltpu.sync_copy(b_dst[r].at[pl.ds(0, fast)],
                        bdst_hbm.at[pl.ds(segbase, fast)])
        nsub = (cnt_s + (sub - 1)) // sub

        def tail_body(b, c2):
          pltpu.sync_copy(b_src[r].at[pl.ds(b * sub, sub)],
                          bsrc_hbm.at[pl.ds(segbase + b * sub, sub)])
          pltpu.sync_copy(b_dst[r].at[pl.ds(b * sub, sub)],
                          bdst_hbm.at[pl.ds(segbase + b * sub, sub)])
          return c2

        lax.fori_loop(fast // sub, nsub, tail_body, 0)
        cnts_row = cnts_row + jnp.where(iota == r, offs[r], 0)

      c_buf[...] = cnts_row
      pltpu.sync_copy(c_buf, cnt_hbm.at[pl.ds(seg * _LANES, _LANES)])
      return carry

    lax.fori_loop(0, nchunk, chunk_body, 0)

  return binning


def _make_agg_kernel(N, D, R, nseg, cap, dpt, sub, fast):
  ndb = D // dpt
  rounds = (R * ndb) // _NW
  assert rounds * _NW == R * ndb and fast % sub == 0

  @functools.partial(
      pl.kernel,
      mesh=_make_mesh(),
      compiler_params=pltpu.CompilerParams(needs_layout_passes=False),
      out_type=[
          jax.ShapeDtypeStruct((R, D, N), jnp.float32),
          jax.ShapeDtypeStruct((R, N), jnp.float32),
      ],
      scratch_types=[pltpu.VMEM((N,), jnp.float32) for _ in range(2 * dpt)] + [
          pltpu.VMEM((N,), jnp.float32),
          pltpu.VMEM((nseg * _LANES,), jnp.int32),
      ] + [pltpu.VMEM((fast,), jnp.int32) for _ in range(4)] + [
          pltpu.VMEM((sub,), jnp.int32),
          pltpu.VMEM((sub,), jnp.int32),
          pltpu.SemaphoreType.DMA,
          pltpu.SemaphoreType.DMA,
      ],
  )
  def agg(xt_hbm, bsrc_hbm, bdst_hbm, cnt_hbm, acct_hbm, cntf_hbm, *rest):
    xblk = rest[:dpt]
    acc = rest[dpt:2 * dpt]
    (cntacc, cnts_all, es0, es1, ed0, ed1, t_src, t_dst,
     sem0, sem1) = rest[2 * dpt:]
    e_src = (es0, es1)
    e_dst = (ed0, ed1)
    sems = (sem0, sem1)
    wid = lax.axis_index("s") * _NC + lax.axis_index("c")
    iota = lax.broadcasted_iota(jnp.int32, (_LANES,), 0)
    zeros16 = jnp.zeros((_LANES,), jnp.float32)
    ones16 = jnp.ones((_LANES,), jnp.float32)
    pltpu.sync_copy(cnt_hbm, cnts_all)

    def round_body(ro, carry):
      t = ro * _NW + wid
      dblock = t // R
      rr = (t % R + dblock) % R
      for f in range(dpt):
        pltpu.sync_copy(xt_hbm.at[dblock * dpt + f], xblk[f])

      @plsc.parallel_loop(0, N // _LANES, unroll=4)
      def _z(i):
        for f in range(dpt):
          acc[f][pl.ds(i * _LANES, _LANES)] = zeros16
        cntacc[pl.ds(i * _LANES, _LANES)] = zeros16

      def start(sg, slot):
        segbase = (sg * R + rr) * cap
        pltpu.async_copy(bsrc_hbm.at[pl.ds(segbase, fast)],
                         e_src[slot], sems[slot])
        pltpu.async_copy(bdst_hbm.at[pl.ds(segbase, fast)],
                         e_dst[slot], sems[slot])

      def wait(slot):
        pltpu.make_async_copy(bsrc_hbm.at[pl.ds(0, fast)],
                              e_src[slot], sems[slot]).wait()
        pltpu.make_async_copy(bdst_hbm.at[pl.ds(0, fast)],
                              e_dst[slot], sems[slot]).wait()

      def vreg_batch(sv_ref, dv_ref, v, base):
        sv = sv_ref[pl.ds(base + v * _LANES, _LANES)]
        dv = dv_ref[pl.ds(base + v * _LANES, _LANES)]
        for f in range(dpt):
          vals = plsc.load_gather(xblk[f], [sv])
          plsc.addupdate_scatter(acc[f], [dv], vals)
        plsc.addupdate_scatter(cntacc, [dv], ones16)

      start(0, 0)
      start(1, 1)

      def pair_body(i, c2):
        for par in range(2):
          sg = i * 2 + par
          wait(par)
          row = cnts_all[pl.ds(sg * _LANES, _LANES)]
          cnt = jnp.sum(jnp.where(iota == rr, row, 0))
          nv = (jnp.minimum(cnt, fast) + (_LANES - 1)) // _LANES

          @plsc.parallel_loop(0, nv, unroll=4)
          def _v(v):
            vreg_batch(e_src[par], e_dst[par], v, 0)

          nsubchunks = (cnt + (sub - 1)) // sub
          segbase = (sg * R + rr) * cap

          def tail_body(b, c3):
            pltpu.sync_copy(bsrc_hbm.at[pl.ds(segbase + b * sub, sub)], t_src)
            pltpu.sync_copy(bdst_hbm.at[pl.ds(segbase + b * sub, sub)], t_dst)

            @plsc.parallel_loop(0, sub // _LANES, unroll=4)
            def _tv(v):
              vreg_batch(t_src, t_dst, v, 0)
            return c3

          lax.fori_loop(fast // sub, nsubchunks, tail_body, 0)
          start(sg + 2, par)
        return c2

      lax.fori_loop(0, nseg // 2, pair_body, 0)
      wait(0)
      wait(1)

      for f in range(dpt):
        pltpu.sync_copy(acc[f], acct_hbm.at[rr, dblock * dpt + f])

      @pl.when(dblock == 0)
      def _():
        pltpu.sync_copy(cntacc, cntf_hbm.at[rr])

      return carry

    lax.fori_loop(0, rounds, round_body, 0)

  return agg


def _make_combine_kernel(N, D, R, bn):
  assert N % bn == 0

  def body(x_ref, acc_ref, cnt_ref, w_ref, wr_ref, b_ref, out_ref):
    x = x_ref[...]
    out = jnp.dot(x, wr_ref[...], preferred_element_type=jnp.float32)
    out = out + b_ref[...]
    for r in range(R):
      a = acc_ref[r]
      c = cnt_ref[pl.ds(r, 1), :]
      inv = 1.0 / jnp.maximum(c, 1.0)
      out = out + lax.dot_general(
          a * inv, w_ref[r], (((0,), (0,)), ((), ())),
          preferred_element_type=jnp.float32)
    out_ref[...] = jnp.maximum(out, 0.0)

  return pl.pallas_call(
      body,
      grid=(N // bn,),
      in_specs=[
          pl.BlockSpec((bn, D), lambda i: (i, 0)),
          pl.BlockSpec((R, D, bn), lambda i: (0, 0, i)),
          pl.BlockSpec((R, bn), lambda i: (0, i)),
          pl.BlockSpec((R, D, D), lambda i: (0, 0, 0)),
          pl.BlockSpec((D, D), lambda i: (0, 0)),
          pl.BlockSpec((1, D), lambda i: (0, 0)),
      ],
      out_specs=pl.BlockSpec((bn, D), lambda i: (i, 0)),
      out_shape=jax.ShapeDtypeStruct((N, D), jnp.float32),
  )


def kernel(node_features, edge_index, edge_type, W, W_root, bias):
  N, D = node_features.shape
  E = edge_index.shape[1]
  R = W.shape[0]

  chunk = 5000
  nchunk = (E // _NW) // chunk
  cap = 5632
  nseg = _NW * nchunk
  dpt = 4
  sub = 512
  fast = 1024
  bn = 1024
  npad = ((N + bn - 1) // bn) * bn

  src = edge_index[0]
  dst = edge_index[1]
  xp = jnp.pad(node_features, ((0, npad - N), (0, 0)))
  xt = xp.T

  binning = _make_binning_kernel(E, R, chunk, nchunk, cap, sub, fast, N)
  bsrc, bdst, cnts = binning(src, dst, edge_type)

  agg = _make_agg_kernel(npad, D, R, nseg, cap, dpt, sub, fast)
  acct, cntf = agg(xt, bsrc, bdst, cnts)

  combine = _make_combine_kernel(npad, D, R, bn)
  out = combine(xp, acct, cntf, W, W_root,
                bias.reshape(1, D).astype(jnp.float32))
  return out[:N]


# speedup vs baseline: 19.3808x; 1.0056x over previous
import functools

import jax
import jax.numpy as jnp
from jax import lax
from jax.experimental import pallas as pl
from jax.experimental.pallas import tpu as pltpu
from jax.experimental.pallas import tpu_sc as plsc

_NC = 2
_NS = 16
_NW = _NC * _NS
_LANES = 16


def _make_mesh():
  return plsc.VectorSubcoreMesh(core_axis_name="c", subcore_axis_name="s",
                                num_cores=_NC, num_subcores=_NS)


def _make_binning_kernel(E, R, chunk, nchunk, cap, sub, fast, dump):
  ew = E // _NW
  assert ew * _NW == E and chunk * nchunk == ew
  assert fast % sub == 0 and cap >= chunk + sub and cap % sub == 0
  nseg = _NW * nchunk
  buf = chunk + sub

  @functools.partial(
      pl.kernel,
      mesh=_make_mesh(),
      compiler_params=pltpu.CompilerParams(needs_layout_passes=False),
      out_type=[
          jax.ShapeDtypeStruct(((nseg + 2) * R * cap,), jnp.int32),
          jax.ShapeDtypeStruct(((nseg + 2) * R * cap,), jnp.int32),
          jax.ShapeDtypeStruct((nseg * _LANES,), jnp.int32),
      ],
      scratch_types=[
          pltpu.VMEM((ew + _LANES,), jnp.int32),
          pltpu.VMEM((ew + _LANES,), jnp.int32),
          pltpu.VMEM((ew + _LANES,), jnp.int32),
      ] + [pltpu.VMEM((buf,), jnp.int32) for _ in range(2 * R)] + [
          pltpu.VMEM((_LANES,), jnp.int32),
      ],
  )
  def binning(src_hbm, dst_hbm, et_hbm, bsrc_hbm, bdst_hbm, cnt_hbm,
              s_src, s_dst, s_et, *rest):
    b_src = rest[:R]
    b_dst = rest[R:2 * R]
    c_buf = rest[2 * R]
    wid = lax.axis_index("s") * _NC + lax.axis_index("c")
    iota = lax.broadcasted_iota(jnp.int32, (_LANES,), 0)
    zero16 = jnp.zeros((_LANES,), jnp.int32)
    dump16 = jnp.full((_LANES,), dump, jnp.int32)
    base = wid * ew
    pltpu.sync_copy(src_hbm.at[pl.ds(base, ew)], s_src.at[pl.ds(0, ew)])
    pltpu.sync_copy(dst_hbm.at[pl.ds(base, ew)], s_dst.at[pl.ds(0, ew)])
    pltpu.sync_copy(et_hbm.at[pl.ds(base, ew)], s_et.at[pl.ds(0, ew)])
    nvec = (chunk + _LANES - 1) // _LANES

    def chunk_body(c, carry):
      seg = wid * nchunk + c
      cb = c * chunk

      @plsc.parallel_loop(0, nvec, unroll=2,
                          carry=tuple(jnp.int32(0) for _ in range(R)))
      def offs(v, offs_c):
        t16 = s_et[pl.ds(cb + v * _LANES, _LANES)]
        sv = s_src[pl.ds(cb + v * _LANES, _LANES)]
        dv = s_dst[pl.ds(cb + v * _LANES, _LANES)]
        valid = (v * _LANES + iota) < chunk
        new_offs = []
        for r in range(R):
          m = jnp.logical_and(t16 == r, valid)
          mi = m.astype(jnp.int32)
          incl = plsc.cumsum(mi)
          pos = incl + (offs_c[r] - 1)
          plsc.store_scatter(b_src[r], [pos], sv, mask=m)
          plsc.store_scatter(b_dst[r], [pos], dv, mask=m)
          new_offs.append(offs_c[r] + jnp.sum(mi))
        return tuple(new_offs)

      cnts_row = zero16
      for r in range(R):
        cnt_s = offs[r]
        for k in range(sub // _LANES):
          b_src[r][pl.ds(cnt_s + k * _LANES, _LANES)] = zero16
          b_dst[r][pl.ds(cnt_s + k * _LANES, _LANES)] = dump16
        segbase = (seg * R + r) * cap
        pltpu.sync_copy(b_src[r].at[pl.ds(0, fast)],
                        bsrc_hbm.at[pl.ds(segbase, fast)])
        pltpu.sync_copy(b_dst[r].at[pl.ds(0, fast)],
                        bdst_hbm.at[pl.ds(segbase, fast)])
        nsub = (cnt_s + (sub - 1)) // sub

        def tail_body(b, c2):
          pltpu.sync_copy(b_src[r].at[pl.ds(b * sub, sub)],
                          bsrc_hbm.at[pl.ds(segbase + b * sub, sub)])
          pltpu.sync_copy(b_dst[r].at[pl.ds(b * sub, sub)],
                          bdst_hbm.at[pl.ds(segbase + b * sub, sub)])
          return c2

        lax.fori_loop(fast // sub, nsub, tail_body, 0)
        cnts_row = cnts_row + jnp.where(iota == r, offs[r], 0)

      c_buf[...] = cnts_row
      pltpu.sync_copy(c_buf, cnt_hbm.at[pl.ds(seg * _LANES, _LANES)])
      return carry

    lax.fori_loop(0, nchunk, chunk_body, 0)

  return binning


def _make_agg_kernel(N, D, R, nseg, cap, dpt, sub, fast):
  ndb = D // dpt
  rounds = (R * ndb) // _NW
  assert rounds * _NW == R * ndb and fast % sub == 0

  @functools.partial(
      pl.kernel,
      mesh=_make_mesh(),
      compiler_params=pltpu.CompilerParams(needs_layout_passes=False),
      out_type=[
          jax.ShapeDtypeStruct((R, D, N), jnp.float32),
          jax.ShapeDtypeStruct((R, N), jnp.float32),
      ],
      scratch_types=[pltpu.VMEM((N,), jnp.float32) for _ in range(2 * dpt)] + [
          pltpu.VMEM((N,), jnp.float32),
          pltpu.VMEM((nseg * _LANES,), jnp.int32),
      ] + [pltpu.VMEM((fast,), jnp.int32) for _ in range(4)] + [
          pltpu.VMEM((sub,), jnp.int32),
          pltpu.VMEM((sub,), jnp.int32),
          pltpu.SemaphoreType.DMA,
          pltpu.SemaphoreType.DMA,
      ],
  )
  def agg(xt_hbm, bsrc_hbm, bdst_hbm, cnt_hbm, acct_hbm, cntf_hbm, *rest):
    xblk = rest[:dpt]
    acc = rest[dpt:2 * dpt]
    (cntacc, cnts_all, es0, es1, ed0, ed1, t_src, t_dst,
     sem0, sem1) = rest[2 * dpt:]
    e_src = (es0, es1)
    e_dst = (ed0, ed1)
    sems = (sem0, sem1)
    wid = lax.axis_index("s") * _NC + lax.axis_index("c")
    iota = lax.broadcasted_iota(jnp.int32, (_LANES,), 0)
    zeros16 = jnp.zeros((_LANES,), jnp.float32)
    ones16 = jnp.ones((_LANES,), jnp.float32)
    pltpu.sync_copy(cnt_hbm, cnts_all)

    def round_body(ro, carry):
      t = ro * _NW + wid
      dblock = t // R
      rr = (t % R + dblock) % R
      for f in range(dpt):
        pltpu.sync_copy(xt_hbm.at[dblock * dpt + f], xblk[f])

      @plsc.parallel_loop(0, N // _LANES, unroll=4)
      def _z(i):
        for f in range(dpt):
          acc[f][pl.ds(i * _LANES, _LANES)] = zeros16
        cntacc[pl.ds(i * _LANES, _LANES)] = zeros16

      def start(sg, slot):
        segbase = (sg * R + rr) * cap
        pltpu.async_copy(bsrc_hbm.at[pl.ds(segbase, fast)],
                         e_src[slot], sems[slot])
        pltpu.async_copy(bdst_hbm.at[pl.ds(segbase, fast)],
                         e_dst[slot], sems[slot])

      def wait(slot):
        pltpu.make_async_copy(bsrc_hbm.at[pl.ds(0, fast)],
                              e_src[slot], sems[slot]).wait()
        pltpu.make_async_copy(bdst_hbm.at[pl.ds(0, fast)],
                              e_dst[slot], sems[slot]).wait()

      def vreg_batch(sv_ref, dv_ref, v, with_cnt):
        sv = sv_ref[pl.ds(v * _LANES, _LANES)]
        dv = dv_ref[pl.ds(v * _LANES, _LANES)]
        for f in range(dpt):
          vals = plsc.load_gather(xblk[f], [sv])
          plsc.addupdate_scatter(acc[f], [dv], vals)
        if with_cnt:
          plsc.addupdate_scatter(cntacc, [dv], ones16)

      def make_pair_body(with_cnt):
        def pair_body(i, c2):
          for par in range(2):
            sg = i * 2 + par
            wait(par)
            row = cnts_all[pl.ds(sg * _LANES, _LANES)]
            cnt = jnp.sum(jnp.where(iota == rr, row, 0))
            nv = (jnp.minimum(cnt, fast) + (_LANES - 1)) // _LANES

            @plsc.parallel_loop(0, nv, unroll=4)
            def _v(v):
              vreg_batch(e_src[par], e_dst[par], v, with_cnt)

            nsubchunks = (cnt + (sub - 1)) // sub
            segbase = (sg * R + rr) * cap

            def tail_body(b, c3):
              pltpu.sync_copy(bsrc_hbm.at[pl.ds(segbase + b * sub, sub)], t_src)
              pltpu.sync_copy(bdst_hbm.at[pl.ds(segbase + b * sub, sub)], t_dst)

              @plsc.parallel_loop(0, sub // _LANES, unroll=4)
              def _tv(v):
                vreg_batch(t_src, t_dst, v, with_cnt)
              return c3

            lax.fori_loop(fast // sub, nsubchunks, tail_body, 0)
            start(sg + 2, par)
          return c2
        return pair_body

      start(0, 0)
      start(1, 1)

      @pl.when(dblock == 0)
      def _():
        lax.fori_loop(0, nseg // 2, make_pair_body(True), 0)

      @pl.when(dblock != 0)
      def _():
        lax.fori_loop(0, nseg // 2, make_pair_body(False), 0)

      wait(0)
      wait(1)

      for f in range(dpt):
        pltpu.sync_copy(acc[f], acct_hbm.at[rr, dblock * dpt + f])

      @pl.when(dblock == 0)
      def _():
        pltpu.sync_copy(cntacc, cntf_hbm.at[rr])

      return carry

    lax.fori_loop(0, rounds, round_body, 0)

  return agg


def _make_combine_kernel(N, D, R, bn):
  assert N % bn == 0

  def body(x_ref, acc_ref, cnt_ref, w_ref, wr_ref, b_ref, out_ref):
    x = x_ref[...]
    out = jnp.dot(x, wr_ref[...], preferred_element_type=jnp.float32)
    out = out + b_ref[...]
    for r in range(R):
      a = acc_ref[r]
      c = cnt_ref[pl.ds(r, 1), :]
      inv = 1.0 / jnp.maximum(c, 1.0)
      out = out + lax.dot_general(
          a * inv, w_ref[r], (((0,), (0,)), ((), ())),
          preferred_element_type=jnp.float32)
    out_ref[...] = jnp.maximum(out, 0.0)

  return pl.pallas_call(
      body,
      grid=(N // bn,),
      in_specs=[
          pl.BlockSpec((bn, D), lambda i: (i, 0)),
          pl.BlockSpec((R, D, bn), lambda i: (0, 0, i)),
          pl.BlockSpec((R, bn), lambda i: (0, i)),
          pl.BlockSpec((R, D, D), lambda i: (0, 0, 0)),
          pl.BlockSpec((D, D), lambda i: (0, 0)),
          pl.BlockSpec((1, D), lambda i: (0, 0)),
      ],
      out_specs=pl.BlockSpec((bn, D), lambda i: (i, 0)),
      out_shape=jax.ShapeDtypeStruct((N, D), jnp.float32),
  )


def kernel(node_features, edge_index, edge_type, W, W_root, bias):
  N, D = node_features.shape
  E = edge_index.shape[1]
  R = W.shape[0]

  chunk = 5000
  nchunk = (E // _NW) // chunk
  cap = 5632
  nseg = _NW * nchunk
  dpt = 4
  sub = 512
  fast = 1024
  bn = 1024
  npad = ((N + bn - 1) // bn) * bn

  src = edge_index[0]
  dst = edge_index[1]
  xp = jnp.pad(node_features, ((0, npad - N), (0, 0)))
  xt = xp.T

  binning = _make_binning_kernel(E, R, chunk, nchunk, cap, sub, fast, N)
  bsrc, bdst, cnts = binning(src, dst, edge_type)

  agg = _make_agg_kernel(npad, D, R, nseg, cap, dpt, sub, fast)
  acct, cntf = agg(xt, bsrc, bdst, cnts)

  combine = _make_combine_kernel(npad, D, R, bn)
  out = combine(xp, acct, cntf, W, W_root,
                bias.reshape(1, D).astype(jnp.float32))
  return out[:N]
